# baseline (speedup 1.0000x reference)
import functools

import jax
import jax.numpy as jnp
from jax.experimental import pallas as pl
from jax.experimental.pallas import tpu as pltpu

KNN_K = 4
BN_EPS = 1e-5


def _graph_stage_kernel(x_ref, u_ref):
    x = x_ref[0]
    n = x.shape[0]
    sq = jnp.sum(x * x, axis=1)
    g = jax.lax.dot_general(x, x, (((1,), (1,)), ((), ())),
                            precision=jax.lax.Precision.DEFAULT,
                            preferred_element_type=jnp.float32)
    col = jax.lax.broadcasted_iota(jnp.int32, (n, n), 1)
    row = jax.lax.broadcasted_iota(jnp.int32, (n, n), 0)
    d2 = sq[:, None] + sq[None, :] - 2.0 * g
    d2 = d2 + jnp.where(col == row, jnp.float32(1e12), jnp.float32(0.0))

    a = jnp.zeros((n, n), jnp.float32)
    for _ in range(KNN_K):
        m = jnp.min(d2, axis=1, keepdims=True)
        idx = jnp.min(jnp.where(d2 == m, col, n), axis=1, keepdims=True)
        mask = col == idx
        a = a + mask.astype(jnp.float32)
        d2 = jnp.where(mask, jnp.float32(3e38), d2)

    cnt = jnp.sum(a, axis=0)
    ew = 1.0 / jnp.maximum(cnt, 1.0)
    deg = 1.0 + jnp.sum(a * ew[None, :], axis=1)
    dinv = jax.lax.rsqrt(deg)
    t = jnp.sum(a * dinv[:, None], axis=0)
    s = dinv * ew * t + dinv * dinv
    xb = x.astype(jnp.bfloat16).astype(jnp.float32)
    u_ref[0, 0, :] = jnp.sum(xb * s[:, None], axis=0) * jnp.float32(1.0 / n)


def _head_kernel(u_ref, conv_w_ref, conv_b_ref, comb_ref,
                 w0_ref, g0_ref, b0_ref, w1_ref, g1_ref, b1_ref, w2_ref,
                 out_ref, gf_ref):
    k_filters = conv_w_ref.shape[0]
    wc = jnp.zeros(conv_w_ref.shape[1:], jnp.float32)
    bc = jnp.zeros((conv_b_ref.shape[1],), jnp.float32)
    for k in range(k_filters):
        cw = comb_ref[0, k]
        wck = conv_w_ref[k].astype(jnp.bfloat16).astype(jnp.float32)
        wc = wc + wck * cw
        bc = bc + conv_b_ref[k, :] * cw
    gf = jnp.dot(u_ref[...], wc, precision=jax.lax.Precision.HIGHEST,
                 preferred_element_type=jnp.float32) + bc[None, :]
    gf_ref[...] = gf
    inv_s = jnp.float32(1.0 / (1.0 + BN_EPS) ** 0.5)
    h0 = jnp.dot(gf, w0_ref[...], precision=jax.lax.Precision.DEFAULT,
                 preferred_element_type=jnp.float32)
    x0 = jax.nn.relu(h0 * inv_s * g0_ref[0, :] + b0_ref[0, :] + gf)
    h1 = jnp.dot(x0, w1_ref[...], precision=jax.lax.Precision.DEFAULT,
                 preferred_element_type=jnp.float32)
    x1 = jax.nn.relu(h1 * inv_s * g1_ref[0, :] + b1_ref[0, :] + x0)
    out_ref[...] = jnp.dot(x1, w2_ref[...], precision=jax.lax.Precision.DEFAULT,
                           preferred_element_type=jnp.float32)


@jax.jit
def kernel(Fet, conv_W, conv_b, comb_weight, mlp_W0, bn0_g, bn0_b,
           mlp_W1, bn1_g, bn1_b, mlp_W2):
    b, n, f_in = Fet.shape
    u = pl.pallas_call(
        _graph_stage_kernel,
        grid=(b,),
        in_specs=[pl.BlockSpec((1, n, f_in), lambda i: (i, 0, 0))],
        out_specs=pl.BlockSpec((1, 1, f_in), lambda i: (i, 0, 0)),
        out_shape=jax.ShapeDtypeStruct((b, 1, f_in), jnp.float32),
        compiler_params=pltpu.CompilerParams(
            dimension_semantics=("arbitrary",)),
    )(Fet)
    u = u.reshape(b, f_in)

    out, gf = pl.pallas_call(
        _head_kernel,
        out_shape=(
            jax.ShapeDtypeStruct((b, mlp_W2.shape[1]), jnp.float32),
            jax.ShapeDtypeStruct((b, mlp_W0.shape[1]), jnp.float32),
        ),
    )(u, conv_W, conv_b, comb_weight.reshape(1, -1),
      mlp_W0, bn0_g.reshape(1, -1), bn0_b.reshape(1, -1),
      mlp_W1, bn1_g.reshape(1, -1), bn1_b.reshape(1, -1), mlp_W2)
    return (out, gf)


# speedup vs baseline: 1.1295x; 1.1295x over previous
import jax
import jax.numpy as jnp
from jax.experimental import pallas as pl
from jax.experimental.pallas import tpu as pltpu

KNN_K = 4
BN_EPS = 1e-5


def _fused_kernel(x_ref, conv_w_ref, conv_b_ref, comb_ref,
                  w0_ref, g0_ref, b0_ref, w1_ref, g1_ref, b1_ref, w2_ref,
                  out_ref, gf_ref, u_scr):
    i = pl.program_id(0)
    nsteps = pl.num_programs(0)
    x = x_ref[0]
    n = x.shape[0]
    sq = jnp.sum(x * x, axis=1)
    g = jax.lax.dot_general(x, x, (((1,), (1,)), ((), ())),
                            precision=jax.lax.Precision.DEFAULT,
                            preferred_element_type=jnp.float32)
    col = jax.lax.broadcasted_iota(jnp.int32, (n, n), 1)
    row = jax.lax.broadcasted_iota(jnp.int32, (n, n), 0)
    d2 = sq[:, None] + sq[None, :] - 2.0 * g
    d2 = jnp.where(col == row, jnp.float32(3e38), d2)

    a = jnp.zeros((n, n), jnp.float32)
    for _ in range(KNN_K):
        idx = jnp.argmin(d2, axis=1)
        mask = col == idx[:, None]
        a = a + mask.astype(jnp.float32)
        d2 = jnp.where(mask, jnp.float32(3e38), d2)

    cnt = jnp.sum(a, axis=0)
    ew = 1.0 / jnp.maximum(cnt, 1.0)
    deg = 1.0 + jnp.sum(a * ew[None, :], axis=1)
    dinv = jax.lax.rsqrt(deg)
    t = jnp.sum(a * dinv[:, None], axis=0)
    s = dinv * ew * t + dinv * dinv
    xb = x.astype(jnp.bfloat16).astype(jnp.float32)
    u_scr[pl.ds(i, 1), :] = (jnp.sum(xb * s[:, None], axis=0)
                             * jnp.float32(1.0 / n))[None, :]

    @pl.when(i == nsteps - 1)
    def _head():
        k_filters = conv_w_ref.shape[0]
        wc = jnp.zeros(conv_w_ref.shape[1:], jnp.float32)
        bc = jnp.zeros((conv_b_ref.shape[1],), jnp.float32)
        for k in range(k_filters):
            cw = comb_ref[0, k]
            wck = conv_w_ref[k].astype(jnp.bfloat16).astype(jnp.float32)
            wc = wc + wck * cw
            bc = bc + conv_b_ref[k, :] * cw
        gf = jnp.dot(u_scr[...], wc, precision=jax.lax.Precision.HIGHEST,
                     preferred_element_type=jnp.float32) + bc[None, :]
        gf_ref[...] = gf
        inv_s = jnp.float32(1.0 / (1.0 + BN_EPS) ** 0.5)
        h0 = jnp.dot(gf, w0_ref[...], precision=jax.lax.Precision.DEFAULT,
                     preferred_element_type=jnp.float32)
        x0 = jax.nn.relu(h0 * inv_s * g0_ref[0, :] + b0_ref[0, :] + gf)
        h1 = jnp.dot(x0, w1_ref[...], precision=jax.lax.Precision.DEFAULT,
                     preferred_element_type=jnp.float32)
        x1 = jax.nn.relu(h1 * inv_s * g1_ref[0, :] + b1_ref[0, :] + x0)
        out_ref[...] = jnp.dot(x1, w2_ref[...],
                               precision=jax.lax.Precision.DEFAULT,
                               preferred_element_type=jnp.float32)


@jax.jit
def kernel(Fet, conv_W, conv_b, comb_weight, mlp_W0, bn0_g, bn0_b,
           mlp_W1, bn1_g, bn1_b, mlp_W2):
    b, n, f_in = Fet.shape
    nhid = mlp_W0.shape[1]
    ncls = mlp_W2.shape[1]
    k_filters = conv_W.shape[0]

    def whole(shape):
        nd = len(shape)
        return pl.BlockSpec(shape, lambda i, _nd=nd: (0,) * _nd)

    out, gf = pl.pallas_call(
        _fused_kernel,
        grid=(b,),
        in_specs=[
            pl.BlockSpec((1, n, f_in), lambda i: (i, 0, 0)),
            whole((k_filters, f_in, nhid)),
            whole((k_filters, nhid)),
            whole((1, k_filters)),
            whole((nhid, nhid)),
            whole((1, nhid)),
            whole((1, nhid)),
            whole((nhid, nhid)),
            whole((1, nhid)),
            whole((1, nhid)),
            whole((nhid, ncls)),
        ],
        out_specs=(
            whole((b, ncls)),
            whole((b, nhid)),
        ),
        out_shape=(
            jax.ShapeDtypeStruct((b, ncls), jnp.float32),
            jax.ShapeDtypeStruct((b, nhid), jnp.float32),
        ),
        scratch_shapes=[pltpu.VMEM((b, f_in), jnp.float32)],
        compiler_params=pltpu.CompilerParams(
            dimension_semantics=("arbitrary",)),
    )(Fet, conv_W, conv_b, comb_weight.reshape(1, -1),
      mlp_W0, bn0_g.reshape(1, -1), bn0_b.reshape(1, -1),
      mlp_W1, bn1_g.reshape(1, -1), bn1_b.reshape(1, -1), mlp_W2)
    return (out, gf)
